# baseline (speedup 1.0000x reference)
import functools

import jax
import jax.numpy as jnp
from jax import lax
from jax.experimental import pallas as pl
from jax.experimental.pallas import tpu as pltpu

_CONTRACT_L = (((0,), (0,)), ((), ()))


def _embed_kernel(x_ref, xm_ref, w_ref, b_ref, o_ref, *, n):
    w = w_ref[...]
    b = b_ref[...]
    tb = o_ref.shape[0]
    for i in range(tb):
        xb = x_ref[i].astype(jnp.bfloat16)
        y = lax.dot_general(xb, w, _CONTRACT_L,
                            preferred_element_type=jnp.float32)
        o_ref[i, :n, :] = y + b
        xm = xm_ref[i].astype(jnp.bfloat16)
        ym = lax.dot_general(xm, w, _CONTRACT_L,
                             preferred_element_type=jnp.float32)
        o_ref[i, n:, :] = ym + b


def kernel(x, x_mark, weight, bias):
    B, L, N = x.shape
    M = x_mark.shape[2]
    V = N + M
    D = weight.shape[0]

    w_t = jnp.transpose(weight, (1, 0)).astype(jnp.bfloat16)
    b2d = bias.reshape(1, D)

    tb = 4 if B % 4 == 0 else 1
    gb = B // tb

    return pl.pallas_call(
        functools.partial(_embed_kernel, n=N),
        out_shape=jax.ShapeDtypeStruct((B, V, D), x.dtype),
        grid=(gb,),
        in_specs=[
            pl.BlockSpec((tb, L, N), lambda b: (b, 0, 0)),
            pl.BlockSpec((tb, L, M), lambda b: (b, 0, 0)),
            pl.BlockSpec((L, D), lambda b: (0, 0)),
            pl.BlockSpec((1, D), lambda b: (0, 0)),
        ],
        out_specs=pl.BlockSpec((tb, V, D), lambda b: (b, 0, 0)),
        compiler_params=pltpu.CompilerParams(
            dimension_semantics=("parallel",),
            vmem_limit_bytes=48 * 1024 * 1024,
        ),
        cost_estimate=pl.CostEstimate(
            flops=2 * B * V * L * D,
            transcendentals=0,
            bytes_accessed=4 * (B * L * V + B * V * D) + 2 * L * D + 4 * D,
        ),
    )(x, x_mark, w_t, b2d)


# speedup vs baseline: 2.2970x; 2.2970x over previous
import functools

import jax
import jax.numpy as jnp
from jax import lax
from jax.experimental import pallas as pl
from jax.experimental.pallas import tpu as pltpu

_CONTRACT_L = (((0,), (0,)), ((), ()))


def _embed_kernel(x_ref, xm_ref, w_ref, b_ref, o_ref, ym_ref, *, n, m):
    w = w_ref[...]
    b = b_ref[...]
    tb = o_ref.shape[1]
    nb = xm_ref.shape[1] // m
    for i in range(tb):
        xb = x_ref[i].astype(jnp.bfloat16)
        y = lax.dot_general(xb, w, _CONTRACT_L,
                            preferred_element_type=jnp.float32)
        o_ref[:n, i, :] = y + b
    xm = xm_ref[...].astype(jnp.bfloat16)
    ym_ref[...] = lax.dot_general(
        xm, w, _CONTRACT_L, preferred_element_type=jnp.float32) + b
    b0 = pl.program_id(0) * tb
    for j in range(m):
        o_ref[n + j, :, :] = ym_ref[pl.ds(j * nb + b0, tb), :]


def kernel(x, x_mark, weight, bias):
    B, L, N = x.shape
    M = x_mark.shape[2]
    V = N + M
    D = weight.shape[0]

    w_t = jnp.transpose(weight, (1, 0)).astype(jnp.bfloat16)
    b2d = bias.reshape(1, D)
    xm2 = jnp.transpose(x_mark, (1, 2, 0)).reshape(L, M * B)

    tb = 8 if B % 8 == 0 else 1
    gb = B // tb

    out_t = pl.pallas_call(
        functools.partial(_embed_kernel, n=N, m=M),
        out_shape=jax.ShapeDtypeStruct((V, B, D), x.dtype),
        grid=(gb,),
        in_specs=[
            pl.BlockSpec((tb, L, N), lambda b: (b, 0, 0)),
            pl.BlockSpec((L, M * B), lambda b: (0, 0)),
            pl.BlockSpec((L, D), lambda b: (0, 0)),
            pl.BlockSpec((1, D), lambda b: (0, 0)),
        ],
        out_specs=pl.BlockSpec((V, tb, D), lambda b: (0, b, 0)),
        scratch_shapes=[pltpu.VMEM((M * B, D), jnp.float32)],
        compiler_params=pltpu.CompilerParams(
            dimension_semantics=("parallel",),
            vmem_limit_bytes=56 * 1024 * 1024,
        ),
        cost_estimate=pl.CostEstimate(
            flops=2 * B * V * L * D,
            transcendentals=0,
            bytes_accessed=4 * (B * L * V + B * V * D) + 2 * L * D + 4 * D,
        ),
    )(x, xm2, w_t, b2d)
    return jnp.transpose(out_t, (1, 0, 2))
